# baseline (speedup 1.0000x reference)
import functools

import jax
import jax.numpy as jnp
import numpy as np
from jax import lax
from jax.experimental import pallas as pl
from jax.experimental.pallas import tpu as pltpu
from jax.experimental.pallas import tpu_sc as plsc

U, I, R, DEG, D = 512, 512, 1000000, 64, 64
NROWS = U * DEG
GW = 128
PW = 2 * D
HID = 1024
OUT = 128
KD = DEG * (D + D)


def _gather_all(ur_idx, ui_idx, ir_idx, iu_idx, rvp, uvp, ivp):
    mesh = plsc.VectorSubcoreMesh(core_axis_name="core", subcore_axis_name="subcore")
    out_type = [jax.ShapeDtypeStruct((NROWS, PW), jnp.float32) for _ in range(4)]

    @functools.partial(pl.kernel, out_type=out_type, mesh=mesh)
    def sc_kernel(ur_h, ui_h, ir_h, iu_h, rv_h, uv_h, iv_h, o1, o2, o3, o4):
        for idx_h, tab_h, out_h in ((ur_h, rv_h, o1), (ui_h, iv_h, o2),
                                    (ir_h, rv_h, o3), (iu_h, uv_h, o4)):
            def body(i_vmem, o_vmem, tab=tab_h):
                pltpu.sync_copy(tab.at[i_vmem.at[0]], o_vmem)

            pltpu.emit_pipeline(
                body,
                grid=(NROWS // GW,),
                in_specs=[pl.BlockSpec((1, GW), index_map=lambda i: (0, i))],
                out_specs=[pl.BlockSpec((GW, PW), index_map=lambda i: (i, 0))],
                core_axis_name=("core", "subcore"),
                dimension_semantics=(pltpu.PARALLEL,),
            )(idx_h, out_h)

    return sc_kernel(ur_idx, ui_idx, ir_idx, iu_idx, rvp, uvp, ivp)


_NW = 8
_WB = HID // _NW
_KT = 512
_DT = 256


def _tc_body(b1_ref, b2_ref, k1_ref, k2_ref, mc1_ref, mc2_ref, w_ref, wc_ref,
             out_ref, xp_ref, s_ref, p_ref):
    j = pl.program_id(0)

    @pl.when(j == 0)
    def _attend():
        mc1 = mc1_ref[...]
        mc2 = mc2_ref[...]
        for g in range(KD // _KT):
            sl = pl.ds(g * _KT, _KT)
            x1 = b1_ref[:, sl] * k1_ref[:, sl]
            x2 = b2_ref[:, sl] * k2_ref[:, sl]
            xp_ref[:, sl] = (
                jnp.dot(x1, mc1, preferred_element_type=jnp.float32)
                + jnp.dot(x2, mc2, preferred_element_type=jnp.float32)
            ).astype(jnp.bfloat16)
        for kt in range(KD // _KT):
            xk = xp_ref[:, kt * _KT:(kt + 1) * _KT]
            c = lax.dot_general(xk, xk, (((1,), (1,)), ((), ())),
                                preferred_element_type=jnp.float32)
            if kt == 0:
                s_ref[...] = c
            else:
                s_ref[...] += c
        sv = s_ref[...] * (1.0 / np.sqrt(float(KD)))
        m = jnp.max(sv, axis=1, keepdims=True)
        e = jnp.exp(sv - m)
        p_ref[...] = (e / jnp.sum(e, axis=1, keepdims=True)).astype(jnp.bfloat16)
        pv = p_ref[...]
        for nt in range(KD // _KT):
            xt = xp_ref[:, nt * _KT:(nt + 1) * _KT]
            xp_ref[:, nt * _KT:(nt + 1) * _KT] = jnp.dot(
                pv, xt, preferred_element_type=jnp.float32).astype(jnp.bfloat16)

    hb = None
    for kt in range(KD // _DT):
        c = jnp.dot(xp_ref[:, kt * _DT:(kt + 1) * _DT],
                    w_ref[kt * _DT:(kt + 1) * _DT, :].astype(jnp.bfloat16),
                    preferred_element_type=jnp.float32)
        hb = c if hb is None else hb + c
    hb = jnp.maximum(hb, 0.0).astype(jnp.bfloat16)
    c = jnp.dot(hb, wc_ref[...].astype(jnp.bfloat16),
                preferred_element_type=jnp.float32)

    @pl.when(j == 0)
    def _init():
        out_ref[...] = c

    @pl.when(j > 0)
    def _acc():
        out_ref[...] += c


def _tc_pipeline(b1, b2, k1, k2, mc1, mc2, w, wc):
    return pl.pallas_call(
        _tc_body,
        grid=(_NW,),
        in_specs=[
            pl.BlockSpec((U, KD), lambda j: (0, 0)),
            pl.BlockSpec((U, KD), lambda j: (0, 0)),
            pl.BlockSpec((U, KD), lambda j: (0, 0)),
            pl.BlockSpec((U, KD), lambda j: (0, 0)),
            pl.BlockSpec((512, 512), lambda j: (0, 0)),
            pl.BlockSpec((512, 512), lambda j: (0, 0)),
            pl.BlockSpec((KD, _WB), lambda j: (0, j)),
            pl.BlockSpec((_WB, OUT), lambda j: (j, 0)),
        ],
        out_specs=pl.BlockSpec((U, OUT), lambda j: (0, 0)),
        out_shape=jax.ShapeDtypeStruct((U, OUT), jnp.float32),
        scratch_shapes=[
            pltpu.VMEM((U, KD), jnp.bfloat16),
            pltpu.VMEM((U, U), jnp.float32),
            pltpu.VMEM((U, U), jnp.bfloat16),
        ],
        compiler_params=pltpu.CompilerParams(
            dimension_semantics=("arbitrary",)),
    )(b1, b2, k1, k2, mc1, mc2, w, wc)


def _perm_mats():
    sk = jax.random.split(jax.random.key(42), 4)
    perms = [jax.random.permutation(k, D) for k in sk]

    def build(p, half):
        dl = jnp.arange(4)
        j = jnp.arange(D)
        rows_e = (dl[:, None] * PW + p[None, :]).reshape(-1)
        rows_o = (dl[:, None] * PW + D + p[None, :]).reshape(-1)
        cols = (dl[:, None] * PW + half * D + j[None, :]).reshape(-1)
        mat = jnp.zeros((512, 512), jnp.float32)
        mat = mat.at[rows_e, cols].set(1.0)
        mat = mat.at[rows_o, cols].set(1.0)
        return mat.astype(jnp.bfloat16)

    return (build(perms[0], 0), build(perms[1], 1),
            build(perms[2], 0), build(perms[3], 1))


def _parity_mask(adj):
    par = (adj & 1).astype(jnp.bfloat16)
    par_rep = jnp.repeat(par, PW, axis=1)
    c_lo = jnp.tile(jnp.arange(PW) < D, DEG)[None, :]
    return jnp.where(c_lo, 1.0 - par_rep, par_rep)


def kernel(inputs, user_review_adj, user_item_adj, item_review_adj,
           item_user_adj, review_vecs, user_vecs, item_vecs,
           user_weights, item_weights, concate_user_weights,
           concate_item_weights):
    mc_ur, mc_ri, mc_ir, mc_ru = _perm_mats()
    rvp = review_vecs.reshape(R // 2, PW)
    uvp = user_vecs.reshape(U // 2, PW)
    ivp = item_vecs.reshape(I // 2, PW)
    hidx = [(a.reshape(-1) >> 1).reshape(1, NROWS).astype(jnp.int32)
            for a in (user_review_adj, user_item_adj,
                      item_review_adj, item_user_adj)]
    b_ur, b_ri, b_ir, b_ru = _gather_all(*hidx, rvp, uvp, ivp)
    b_ur = b_ur.reshape(U, KD).astype(jnp.bfloat16)
    b_ri = b_ri.reshape(U, KD).astype(jnp.bfloat16)
    b_ir = b_ir.reshape(I, KD).astype(jnp.bfloat16)
    b_ru = b_ru.reshape(I, KD).astype(jnp.bfloat16)
    user_out = _tc_pipeline(b_ur, b_ri,
                            _parity_mask(user_review_adj),
                            _parity_mask(user_item_adj),
                            mc_ur, mc_ri, user_weights, concate_user_weights)
    item_out = _tc_pipeline(b_ir, b_ru,
                            _parity_mask(item_review_adj),
                            _parity_mask(item_user_adj),
                            mc_ir, mc_ru, item_weights, concate_item_weights)
    return user_out, item_out


# speedup vs baseline: 1.0011x; 1.0011x over previous
import functools

import jax
import jax.numpy as jnp
import numpy as np
from jax import lax
from jax.experimental import pallas as pl
from jax.experimental.pallas import tpu as pltpu
from jax.experimental.pallas import tpu_sc as plsc

U, I, R, DEG, D = 512, 512, 1000000, 64, 64
NROWS = U * DEG
GW = 128
PW = 2 * D
HID = 1024
OUT = 128
KD = DEG * (D + D)


def _gather_all(ur_idx, ui_idx, ir_idx, iu_idx, rvp, uvp, ivp):
    mesh = plsc.VectorSubcoreMesh(core_axis_name="core", subcore_axis_name="subcore")
    out_type = [jax.ShapeDtypeStruct((NROWS, PW), jnp.float32) for _ in range(4)]

    @functools.partial(pl.kernel, out_type=out_type, mesh=mesh)
    def sc_kernel(ur_h, ui_h, ir_h, iu_h, rv_h, uv_h, iv_h, o1, o2, o3, o4):
        for idx_h, tab_h, out_h in ((ur_h, rv_h, o1), (ui_h, iv_h, o2),
                                    (ir_h, rv_h, o3), (iu_h, uv_h, o4)):
            def body(i_vmem, o_vmem, tab=tab_h):
                pltpu.sync_copy(tab.at[i_vmem.at[0]], o_vmem)

            pltpu.emit_pipeline(
                body,
                grid=(NROWS // GW,),
                in_specs=[pl.BlockSpec((1, GW), index_map=lambda i: (0, i))],
                out_specs=[pl.BlockSpec((GW, PW), index_map=lambda i: (i, 0))],
                core_axis_name=("core", "subcore"),
                dimension_semantics=(pltpu.PARALLEL,),
            )(idx_h, out_h)

    return sc_kernel(ur_idx, ui_idx, ir_idx, iu_idx, rvp, uvp, ivp)


_NW = 4
_WB = HID // _NW
_KT = 512
_DT = 256


def _attn_body(b1_ref, b2_ref, k1_ref, k2_ref, mc1_ref, mc2_ref,
               a_ref, xp_ref, s_ref):
    mc1 = mc1_ref[...]
    mc2 = mc2_ref[...]
    for g in range(KD // _KT):
        sl = pl.ds(g * _KT, _KT)
        x1 = b1_ref[:, sl] * k1_ref[:, sl]
        x2 = b2_ref[:, sl] * k2_ref[:, sl]
        xp_ref[:, sl] = (
            jnp.dot(x1, mc1, preferred_element_type=jnp.float32)
            + jnp.dot(x2, mc2, preferred_element_type=jnp.float32)
        ).astype(jnp.bfloat16)
    for kt in range(KD // _KT):
        xk = xp_ref[:, kt * _KT:(kt + 1) * _KT]
        c = lax.dot_general(xk, xk, (((1,), (1,)), ((), ())),
                            preferred_element_type=jnp.float32)
        if kt == 0:
            s_ref[...] = c
        else:
            s_ref[...] += c
    sv = s_ref[...] * (1.0 / np.sqrt(float(KD)))
    m = jnp.max(sv, axis=1, keepdims=True)
    e = jnp.exp(sv - m)
    pv = (e / jnp.sum(e, axis=1, keepdims=True)).astype(jnp.bfloat16)
    for nt in range(KD // _KT):
        xt = xp_ref[:, nt * _KT:(nt + 1) * _KT]
        a_ref[:, nt * _KT:(nt + 1) * _KT] = jnp.dot(
            pv, xt, preferred_element_type=jnp.float32).astype(jnp.bfloat16)


def _dense_body(a_ref, w_ref, wc_ref, out_ref):
    j = pl.program_id(0)
    hb = None
    for kt in range(KD // _DT):
        c = jnp.dot(a_ref[:, kt * _DT:(kt + 1) * _DT],
                    w_ref[kt * _DT:(kt + 1) * _DT, :].astype(jnp.bfloat16),
                    preferred_element_type=jnp.float32)
        hb = c if hb is None else hb + c
    hb = jnp.maximum(hb, 0.0).astype(jnp.bfloat16)
    c = jnp.dot(hb, wc_ref[...].astype(jnp.bfloat16),
                preferred_element_type=jnp.float32)

    @pl.when(j == 0)
    def _init():
        out_ref[...] = c

    @pl.when(j > 0)
    def _acc():
        out_ref[...] += c


def _tc_pipeline(b1, b2, k1, k2, mc1, mc2, w, wc):
    a = pl.pallas_call(
        _attn_body,
        in_specs=[
            pl.BlockSpec((U, KD), lambda: (0, 0)),
            pl.BlockSpec((U, KD), lambda: (0, 0)),
            pl.BlockSpec((U, KD), lambda: (0, 0)),
            pl.BlockSpec((U, KD), lambda: (0, 0)),
            pl.BlockSpec((512, 512), lambda: (0, 0)),
            pl.BlockSpec((512, 512), lambda: (0, 0)),
        ],
        out_specs=pl.BlockSpec((U, KD), lambda: (0, 0)),
        out_shape=jax.ShapeDtypeStruct((U, KD), jnp.bfloat16),
        scratch_shapes=[
            pltpu.VMEM((U, KD), jnp.bfloat16),
            pltpu.VMEM((U, U), jnp.float32),
        ],
    )(b1, b2, k1, k2, mc1, mc2)
    return pl.pallas_call(
        _dense_body,
        grid=(_NW,),
        in_specs=[
            pl.BlockSpec((U, KD), lambda j: (0, 0)),
            pl.BlockSpec((KD, _WB), lambda j: (0, j)),
            pl.BlockSpec((_WB, OUT), lambda j: (j, 0)),
        ],
        out_specs=pl.BlockSpec((U, OUT), lambda j: (0, 0)),
        out_shape=jax.ShapeDtypeStruct((U, OUT), jnp.float32),
        compiler_params=pltpu.CompilerParams(
            dimension_semantics=("arbitrary",)),
    )(a, w, wc)


def _perm_mats():
    sk = jax.random.split(jax.random.key(42), 4)
    perms = [jax.random.permutation(k, D) for k in sk]

    def build(p, half):
        dl = jnp.arange(4)
        j = jnp.arange(D)
        rows_e = (dl[:, None] * PW + p[None, :]).reshape(-1)
        rows_o = (dl[:, None] * PW + D + p[None, :]).reshape(-1)
        cols = (dl[:, None] * PW + half * D + j[None, :]).reshape(-1)
        mat = jnp.zeros((512, 512), jnp.float32)
        mat = mat.at[rows_e, cols].set(1.0)
        mat = mat.at[rows_o, cols].set(1.0)
        return mat.astype(jnp.bfloat16)

    return (build(perms[0], 0), build(perms[1], 1),
            build(perms[2], 0), build(perms[3], 1))


def _parity_mask(adj):
    par = (adj & 1).astype(jnp.bfloat16)
    par_rep = jnp.repeat(par, PW, axis=1)
    c_lo = jnp.tile(jnp.arange(PW) < D, DEG)[None, :]
    return jnp.where(c_lo, 1.0 - par_rep, par_rep)


def kernel(inputs, user_review_adj, user_item_adj, item_review_adj,
           item_user_adj, review_vecs, user_vecs, item_vecs,
           user_weights, item_weights, concate_user_weights,
           concate_item_weights):
    mc_ur, mc_ri, mc_ir, mc_ru = _perm_mats()
    rvp = review_vecs.reshape(R // 2, PW)
    uvp = user_vecs.reshape(U // 2, PW)
    ivp = item_vecs.reshape(I // 2, PW)
    hidx = [(a.reshape(-1) >> 1).reshape(1, NROWS).astype(jnp.int32)
            for a in (user_review_adj, user_item_adj,
                      item_review_adj, item_user_adj)]
    b_ur, b_ri, b_ir, b_ru = _gather_all(*hidx, rvp, uvp, ivp)
    b_ur = b_ur.reshape(U, KD).astype(jnp.bfloat16)
    b_ri = b_ri.reshape(U, KD).astype(jnp.bfloat16)
    b_ir = b_ir.reshape(I, KD).astype(jnp.bfloat16)
    b_ru = b_ru.reshape(I, KD).astype(jnp.bfloat16)
    user_out = _tc_pipeline(b_ur, b_ri,
                            _parity_mask(user_review_adj),
                            _parity_mask(user_item_adj),
                            mc_ur, mc_ri, user_weights, concate_user_weights)
    item_out = _tc_pipeline(b_ir, b_ru,
                            _parity_mask(item_review_adj),
                            _parity_mask(item_user_adj),
                            mc_ir, mc_ru, item_weights, concate_item_weights)
    return user_out, item_out


# speedup vs baseline: 1.0425x; 1.0413x over previous
import functools

import jax
import jax.numpy as jnp
import numpy as np
from jax import lax
from jax.experimental import pallas as pl
from jax.experimental.pallas import tpu as pltpu
from jax.experimental.pallas import tpu_sc as plsc

U, I, R, DEG, D = 512, 512, 1000000, 64, 64
NROWS = U * DEG
GW = 128
PW = 2 * D
HID = 1024
OUT = 128
KD = DEG * (D + D)
KH = KD // 2


def _gather_all(ur_idx, ui_idx, ir_idx, iu_idx, rvp, uvp, ivp):
    mesh = plsc.VectorSubcoreMesh(core_axis_name="core", subcore_axis_name="subcore")
    out_type = [jax.ShapeDtypeStruct((NROWS, PW), jnp.float32) for _ in range(4)]

    @functools.partial(pl.kernel, out_type=out_type, mesh=mesh)
    def sc_kernel(ur_h, ui_h, ir_h, iu_h, rv_h, uv_h, iv_h, o1, o2, o3, o4):
        for idx_h, tab_h, out_h in ((ur_h, rv_h, o1), (ui_h, iv_h, o2),
                                    (ir_h, rv_h, o3), (iu_h, uv_h, o4)):
            def body(i_vmem, o_vmem, tab=tab_h):
                pltpu.sync_copy(tab.at[i_vmem.at[0]], o_vmem)

            pltpu.emit_pipeline(
                body,
                grid=(NROWS // GW,),
                in_specs=[pl.BlockSpec((1, GW), index_map=lambda i: (0, i))],
                out_specs=[pl.BlockSpec((GW, PW), index_map=lambda i: (i, 0))],
                core_axis_name=("core", "subcore"),
                dimension_semantics=(pltpu.PARALLEL,),
            )(idx_h, out_h)

    return sc_kernel(ur_idx, ui_idx, ir_idx, iu_idx, rvp, uvp, ivp)


_RB = 128


def _compact_body(b1_ref, b2_ref, p1_ref, p2_ref, x_ref):
    for s in range(2):
        b_ref = b1_ref if s == 0 else b2_ref
        p_ref = p1_ref if s == 0 else p2_ref
        off = s * KH
        for d in range(DEG):
            lo = b_ref[:, d * PW:d * PW + D]
            hi = b_ref[:, d * PW + D:(d + 1) * PW]
            p = p_ref[:, d:d + 1]
            x_ref[:, off + d * D:off + (d + 1) * D] = jnp.where(
                p > 0.5, hi, lo).astype(jnp.bfloat16)


def _compact(b1, b2, p1, p2):
    return pl.pallas_call(
        _compact_body,
        grid=(U // _RB,),
        in_specs=[
            pl.BlockSpec((_RB, KD), lambda i: (i, 0)),
            pl.BlockSpec((_RB, KD), lambda i: (i, 0)),
            pl.BlockSpec((_RB, DEG), lambda i: (i, 0)),
            pl.BlockSpec((_RB, DEG), lambda i: (i, 0)),
        ],
        out_specs=pl.BlockSpec((_RB, KD), lambda i: (i, 0)),
        out_shape=jax.ShapeDtypeStruct((U, KD), jnp.bfloat16),
        compiler_params=pltpu.CompilerParams(
            dimension_semantics=("arbitrary",)),
    )(b1, b2, p1, p2)


_NW = 4
_WB = HID // _NW
_KT = 512


def _fused_body(x_ref, w_ref, wc_ref, out_ref, a_ref, s_ref):
    j = pl.program_id(0)

    @pl.when(j == 0)
    def _attend():
        for t in range(KD // _KT):
            xk = x_ref[:, t * _KT:(t + 1) * _KT]
            c = lax.dot_general(xk, xk, (((1,), (1,)), ((), ())),
                                preferred_element_type=jnp.float32)
            if t == 0:
                s_ref[...] = c
            else:
                s_ref[...] += c
        sv = s_ref[...] * (1.0 / np.sqrt(float(KD)))
        m = jnp.max(sv, axis=1, keepdims=True)
        e = jnp.exp(sv - m)
        pv = (e / jnp.sum(e, axis=1, keepdims=True)).astype(jnp.bfloat16)
        for t in range(KD // _KT):
            a_ref[:, t * _KT:(t + 1) * _KT] = jnp.dot(
                pv, x_ref[:, t * _KT:(t + 1) * _KT],
                preferred_element_type=jnp.float32).astype(jnp.bfloat16)

    hb = None
    for kt in range(KD // _KT):
        c = jnp.dot(a_ref[:, kt * _KT:(kt + 1) * _KT],
                    w_ref[kt * _KT:(kt + 1) * _KT, :],
                    preferred_element_type=jnp.float32)
        hb = c if hb is None else hb + c
    hb = jnp.maximum(hb, 0.0).astype(jnp.bfloat16)
    c = jnp.dot(hb, wc_ref[...].astype(jnp.bfloat16),
                preferred_element_type=jnp.float32)

    @pl.when(j == 0)
    def _init():
        out_ref[...] = c

    @pl.when(j > 0)
    def _acc():
        out_ref[...] += c


def _attn_dense(x, w, wc):
    return pl.pallas_call(
        _fused_body,
        grid=(_NW,),
        in_specs=[
            pl.BlockSpec((U, KD), lambda j: (0, 0)),
            pl.BlockSpec((KD, _WB), lambda j: (0, j)),
            pl.BlockSpec((_WB, OUT), lambda j: (j, 0)),
        ],
        out_specs=pl.BlockSpec((U, OUT), lambda j: (0, 0)),
        out_shape=jax.ShapeDtypeStruct((U, OUT), jnp.float32),
        scratch_shapes=[
            pltpu.VMEM((U, KD), jnp.bfloat16),
            pltpu.VMEM((U, U), jnp.float32),
        ],
        compiler_params=pltpu.CompilerParams(
            dimension_semantics=("arbitrary",)),
    )(x, w, wc)


def _w_rows(p1, p2):
    d = jnp.arange(DEG)
    inv1 = jnp.argsort(p1)
    inv2 = jnp.argsort(p2)
    rows_a = (d[:, None] * PW + inv1[None, :]).reshape(-1)
    rows_b = (d[:, None] * PW + D + inv2[None, :]).reshape(-1)
    return jnp.concatenate([rows_a, rows_b])


def kernel(inputs, user_review_adj, user_item_adj, item_review_adj,
           item_user_adj, review_vecs, user_vecs, item_vecs,
           user_weights, item_weights, concate_user_weights,
           concate_item_weights):
    sk = jax.random.split(jax.random.key(42), 4)
    p_ur, p_ri, p_ir, p_ru = (jax.random.permutation(k, D) for k in sk)
    rvp = review_vecs.reshape(R // 2, PW)
    uvp = user_vecs.reshape(U // 2, PW)
    ivp = item_vecs.reshape(I // 2, PW)
    hidx = [(a.reshape(-1) >> 1).reshape(1, NROWS)
            for a in (user_review_adj, user_item_adj,
                      item_review_adj, item_user_adj)]
    b_ur, b_ri, b_ir, b_ru = _gather_all(*hidx, rvp, uvp, ivp)
    par = [(a & 1).astype(jnp.float32)
           for a in (user_review_adj, user_item_adj,
                     item_review_adj, item_user_adj)]
    x_u = _compact(b_ur.reshape(U, KD), b_ri.reshape(U, KD), par[0], par[1])
    x_i = _compact(b_ir.reshape(I, KD), b_ru.reshape(I, KD), par[2], par[3])
    w_u = user_weights[_w_rows(p_ur, p_ri)].astype(jnp.bfloat16)
    w_i = item_weights[_w_rows(p_ir, p_ru)].astype(jnp.bfloat16)
    user_out = _attn_dense(x_u, w_u, concate_user_weights)
    item_out = _attn_dense(x_i, w_i, concate_item_weights)
    return user_out, item_out
